# baseline (speedup 1.0000x reference)
import functools

import jax
import jax.numpy as jnp
from jax import lax
from jax.experimental import pallas as pl
from jax.experimental.pallas import tpu as pltpu
from jax.experimental.pallas import tpu_sc as plsc

B = 1024
D = 16
BI = 128


def _make_sc_gather():
    info = plsc.get_sparse_core_info()
    nc, ns = info.num_cores, info.num_subcores
    nw = nc * ns
    bpw = B // nw
    mesh = plsc.VectorSubcoreMesh(core_axis_name="c", subcore_axis_name="s")

    @functools.partial(
        pl.kernel,
        out_type=[jax.ShapeDtypeStruct((B, D), jnp.float32)] * 4,
        mesh=mesh,
        scratch_types=[
            pltpu.VMEM((bpw,), jnp.int32),
            pltpu.VMEM((bpw,), jnp.int32),
            pltpu.VMEM((bpw,), jnp.int32),
            pltpu.VMEM((bpw, D), jnp.float32),
            pltpu.VMEM((bpw, D), jnp.float32),
            pltpu.VMEM((bpw, D), jnp.float32),
            pltpu.VMEM((bpw, D), jnp.float32),
            pltpu.SemaphoreType.DMA,
        ],
        compiler_params=pltpu.CompilerParams(use_tc_tiling_on_sc=False),
    )
    def gather_kernel(ent_hbm, rel_hbm, wr_hbm, ih_hbm, ir_hbm, it_hbm,
                      h_out, r_out, t_out, w_out,
                      ih_v, ir_v, it_v, hv, rv, tv, wv, sem):
        wid = lax.axis_index("s") * nc + lax.axis_index("c")
        base = wid * bpw
        pltpu.sync_copy(ih_hbm.at[pl.ds(base, bpw)], ih_v)
        pltpu.sync_copy(ir_hbm.at[pl.ds(base, bpw)], ir_v)
        pltpu.sync_copy(it_hbm.at[pl.ds(base, bpw)], it_v)
        cp_h = pltpu.async_copy(ent_hbm.at[ih_v], hv, sem)
        cp_r = pltpu.async_copy(rel_hbm.at[ir_v], rv, sem)
        cp_t = pltpu.async_copy(ent_hbm.at[it_v], tv, sem)
        cp_w = pltpu.async_copy(wr_hbm.at[ir_v], wv, sem)
        cp_h.wait()
        cp_r.wait()
        cp_t.wait()
        cp_w.wait()
        pltpu.sync_copy(hv, h_out.at[pl.ds(base, bpw)])
        pltpu.sync_copy(rv, r_out.at[pl.ds(base, bpw)])
        pltpu.sync_copy(tv, t_out.at[pl.ds(base, bpw)])
        pltpu.sync_copy(wv, w_out.at[pl.ds(base, bpw)])

    return gather_kernel


def _score_body(h_ref, r_ref, t_ref, w_ref, out_ref):
    h = h_ref[...]
    r = r_ref[...]
    t = t_ref[...]
    w = w_ref[...]
    g = t - h
    d = r - g
    dk = jnp.sum(g, axis=0, keepdims=True)
    w2 = w * w
    b = dk * w2
    v = jnp.sum(w * w2 * dk, axis=1, keepdims=True)
    hi = jax.lax.Precision.HIGHEST
    dot = lambda a, c: jax.lax.dot_general(
        a, c, (((1,), (1,)), ((), ())), precision=hi,
        preferred_element_type=jnp.float32)
    m = dot(w, g)
    p = dot(w, d)
    gg = dot(b, d)
    nd = dot(jnp.ones((1, D), jnp.float32), d * d)
    nb = jnp.sum(b * b, axis=1, keepdims=True)
    ww = jnp.sum(w2, axis=1, keepdims=True)
    q = jnp.sum(w * b, axis=1, keepdims=True)
    u = m - v
    acc = nd + nb + 2.0 * gg + u * u * ww + 2.0 * u * (p + q)
    out_ref[...] = jnp.sqrt(jnp.maximum(acc, 0.0))


def _score(h, r, t, w):
    return pl.pallas_call(
        _score_body,
        grid=(B // BI,),
        in_specs=[
            pl.BlockSpec((B, D), lambda i: (0, 0)),
            pl.BlockSpec((B, D), lambda i: (0, 0)),
            pl.BlockSpec((B, D), lambda i: (0, 0)),
            pl.BlockSpec((BI, D), lambda i: (i, 0)),
        ],
        out_specs=pl.BlockSpec((BI, B), lambda i: (i, 0)),
        out_shape=jax.ShapeDtypeStruct((B, B), jnp.float32),
    )(h, r, t, w)


def kernel(pos_sample, ent_emb, rel_emb, wr_emb):
    idx_h = pos_sample[:, 0]
    idx_r = pos_sample[:, 1]
    idx_t = pos_sample[:, 2]
    h, r, t, w = _make_sc_gather()(ent_emb, rel_emb, wr_emb, idx_h, idx_r, idx_t)
    return _score(h, r, t, w)
